# baseline (speedup 1.0000x reference)
import functools

import jax
import jax.numpy as jnp
import numpy as np
from jax.experimental import pallas as pl

_NQ = 8
_K = 1024
_D = 128
_TT = 512


def _rvq_kernel(x_ref, cb_ref, q_ref, codes_ref, sq_ref):
    xt = x_ref[0]
    r = xt.T
    codes_rows = []
    sq_rows = []
    for i in range(_NQ):
        cb = cb_ref[i]
        c2 = jnp.sum(cb * cb, axis=1)
        r2 = jnp.sum(r * r, axis=1, keepdims=True)
        dots = jax.lax.dot_general(
            r, cb, (((1,), (1,)), ((), ())),
            preferred_element_type=jnp.float32,
            precision=jax.lax.Precision.DEFAULT)
        dist = (r2 - 2.0 * dots) + c2[None, :]
        dmin = jnp.min(dist, axis=1, keepdims=True)
        iota = jax.lax.broadcasted_iota(jnp.int32, (_TT, _K), 1)
        idx = jnp.min(jnp.where(dist <= dmin, iota, _K), axis=1)
        codes_rows.append(idx)
        onehot = (iota == idx[:, None]).astype(jnp.float32)
        q = jax.lax.dot_general(
            onehot, cb, (((1,), (0,)), ((), ())),
            preferred_element_type=jnp.float32,
            precision=jax.lax.Precision.HIGHEST)
        r = r - q
        sq_rows.append(jnp.sum(r * r, axis=0))
    q_ref[0] = xt - r.T
    codes_ref[0] = jnp.stack(codes_rows, axis=0).astype(jnp.int32)
    sq_ref[0, 0] = jnp.stack(sq_rows, axis=0)


def kernel(x, sample_rate, codebooks):
    B, D, T = x.shape
    grid = (B, T // _TT)
    q, codes_bqt, sq = pl.pallas_call(
        _rvq_kernel,
        grid=grid,
        in_specs=[
            pl.BlockSpec((1, D, _TT), lambda b, t: (b, 0, t)),
            pl.BlockSpec((_NQ, _K, _D), lambda b, t: (0, 0, 0)),
        ],
        out_specs=[
            pl.BlockSpec((1, D, _TT), lambda b, t: (b, 0, t)),
            pl.BlockSpec((1, _NQ, _TT), lambda b, t: (b, 0, t)),
            pl.BlockSpec((1, 1, _NQ, _D), lambda b, t: (b, t, 0, 0)),
        ],
        out_shape=[
            jax.ShapeDtypeStruct((B, D, T), jnp.float32),
            jax.ShapeDtypeStruct((B, _NQ, T), jnp.int32),
            jax.ShapeDtypeStruct((B, T // _TT, _NQ, _D), jnp.float32),
        ],
    )(x, codebooks)
    codes = jnp.transpose(codes_bqt, (1, 0, 2))
    commit_loss = jnp.sum(sq) / np.float32(_NQ * B * T * D)
    bw = jnp.asarray(
        _NQ * (np.log2(_K) * jnp.asarray(sample_rate).astype(jnp.float32) / 1000.0),
        dtype=jnp.float32)
    return q, codes, bw, commit_loss


# speedup vs baseline: 1.5662x; 1.5662x over previous
import jax
import jax.numpy as jnp
import numpy as np
from jax.experimental import pallas as pl

_NQ = 8
_K = 1024
_D = 128
_TT = 512


def _rvq_kernel(x_ref, cb_ref, c2_ref, hi_ref, mid_ref, lo_ref,
                q_ref, codes_ref, sq_ref):
    xt = x_ref[0]
    r = xt.T
    codes_rows = []
    sq_rows = []
    iota = jax.lax.broadcasted_iota(jnp.int32, (_TT, _K), 1)
    for i in range(_NQ):
        cb = cb_ref[i]
        c2 = c2_ref[i]
        r2 = jnp.sum(r * r, axis=1, keepdims=True)
        dots = jax.lax.dot_general(
            r, cb, (((1,), (1,)), ((), ())),
            preferred_element_type=jnp.float32,
            precision=jax.lax.Precision.DEFAULT)
        dist = (r2 - 2.0 * dots) + c2[None, :]
        dmin = jnp.min(dist, axis=1, keepdims=True)
        idx = jnp.min(jnp.where(dist <= dmin, iota, _K), axis=1)
        codes_rows.append(idx)
        onehot = (iota == idx[:, None]).astype(jnp.bfloat16)
        q = (jax.lax.dot_general(
                onehot, hi_ref[i], (((1,), (0,)), ((), ())),
                preferred_element_type=jnp.float32)
             + jax.lax.dot_general(
                onehot, mid_ref[i], (((1,), (0,)), ((), ())),
                preferred_element_type=jnp.float32)
             + jax.lax.dot_general(
                onehot, lo_ref[i], (((1,), (0,)), ((), ())),
                preferred_element_type=jnp.float32))
        r = r - q
        sq_rows.append(jnp.sum(r * r, axis=0))
    q_ref[0] = xt - r.T
    codes_ref[0] = jnp.stack(codes_rows, axis=0).astype(jnp.int32)
    sq_ref[0, 0] = jnp.stack(sq_rows, axis=0)


def kernel(x, sample_rate, codebooks):
    B, D, T = x.shape
    grid = (B, T // _TT)
    cb_hi = codebooks.astype(jnp.bfloat16)
    rem1 = codebooks - cb_hi.astype(jnp.float32)
    cb_mid = rem1.astype(jnp.bfloat16)
    cb_lo = (rem1 - cb_mid.astype(jnp.float32)).astype(jnp.bfloat16)
    c2 = jnp.sum(codebooks ** 2, axis=-1)
    full = pl.BlockSpec((_NQ, _K, _D), lambda b, t: (0, 0, 0))
    q, codes_bqt, sq = pl.pallas_call(
        _rvq_kernel,
        grid=grid,
        in_specs=[
            pl.BlockSpec((1, D, _TT), lambda b, t: (b, 0, t)),
            full,
            pl.BlockSpec((_NQ, _K), lambda b, t: (0, 0)),
            full, full, full,
        ],
        out_specs=[
            pl.BlockSpec((1, D, _TT), lambda b, t: (b, 0, t)),
            pl.BlockSpec((1, _NQ, _TT), lambda b, t: (b, 0, t)),
            pl.BlockSpec((1, 1, _NQ, _D), lambda b, t: (b, t, 0, 0)),
        ],
        out_shape=[
            jax.ShapeDtypeStruct((B, D, T), jnp.float32),
            jax.ShapeDtypeStruct((B, _NQ, T), jnp.int32),
            jax.ShapeDtypeStruct((B, T // _TT, _NQ, _D), jnp.float32),
        ],
    )(x, codebooks, c2, cb_hi, cb_mid, cb_lo)
    codes = jnp.transpose(codes_bqt, (1, 0, 2))
    commit_loss = jnp.sum(sq) / np.float32(_NQ * B * T * D)
    bw = jnp.asarray(
        _NQ * (np.log2(_K) * jnp.asarray(sample_rate).astype(jnp.float32) / 1000.0),
        dtype=jnp.float32)
    return q, codes, bw, commit_loss


# speedup vs baseline: 2.0993x; 1.3404x over previous
import jax
import jax.numpy as jnp
import numpy as np
from jax.experimental import pallas as pl

_NQ = 8
_K = 1024
_D = 128
_TT = 512


def _rvq_kernel(x_ref, cb_ref, b0_ref, b1_ref, b2_ref, b3_ref,
                q_ref, codes_ref, sq_ref):
    xt = x_ref[0]
    r = xt.T
    codes_rows = []
    sq_rows = []
    iota = jax.lax.broadcasted_iota(jnp.int32, (_TT, _K), 1)
    for i in range(_NQ):
        cb = cb_ref[i]
        c2 = jnp.sum(cb * cb, axis=1)
        r2 = jnp.sum(r * r, axis=1, keepdims=True)
        dots = jax.lax.dot_general(
            r, cb, (((1,), (1,)), ((), ())),
            preferred_element_type=jnp.float32,
            precision=jax.lax.Precision.DEFAULT)
        dist = (r2 - 2.0 * dots) + c2[None, :]
        dmin = jnp.min(dist, axis=1, keepdims=True)
        idx = jnp.min(jnp.where(dist <= dmin, iota, _K), axis=1)
        codes_rows.append(idx)
        onehot = (iota == idx[:, None]).astype(jnp.bfloat16)

        def pick(plane_ref):
            f = jax.lax.dot_general(
                onehot, plane_ref[i], (((1,), (0,)), ((), ())),
                preferred_element_type=jnp.float32)
            return f.astype(jnp.int32)

        bits = (pick(b0_ref)
                | (pick(b1_ref) << 8)
                | (pick(b2_ref) << 16)
                | (pick(b3_ref) << 24))
        q = jax.lax.bitcast_convert_type(bits, jnp.float32)
        r = r - q
        sq_rows.append(jnp.sum(r * r, axis=0))
    q_ref[0] = xt - r.T
    codes_ref[0] = jnp.stack(codes_rows, axis=0).astype(jnp.int32)
    sq_ref[0, 0] = jnp.stack(sq_rows, axis=0)


def kernel(x, sample_rate, codebooks):
    B, D, T = x.shape
    grid = (B, T // _TT)
    bits = jax.lax.bitcast_convert_type(codebooks, jnp.uint32)
    planes = [((bits >> (8 * k)) & np.uint32(0xFF)).astype(jnp.bfloat16)
              for k in range(4)]
    full = pl.BlockSpec((_NQ, _K, _D), lambda b, t: (0, 0, 0))
    q, codes_bqt, sq = pl.pallas_call(
        _rvq_kernel,
        grid=grid,
        in_specs=[
            pl.BlockSpec((1, D, _TT), lambda b, t: (b, 0, t)),
            full, full, full, full, full,
        ],
        out_specs=[
            pl.BlockSpec((1, D, _TT), lambda b, t: (b, 0, t)),
            pl.BlockSpec((1, _NQ, _TT), lambda b, t: (b, 0, t)),
            pl.BlockSpec((1, 1, _NQ, _D), lambda b, t: (b, t, 0, 0)),
        ],
        out_shape=[
            jax.ShapeDtypeStruct((B, D, T), jnp.float32),
            jax.ShapeDtypeStruct((B, _NQ, T), jnp.int32),
            jax.ShapeDtypeStruct((B, T // _TT, _NQ, _D), jnp.float32),
        ],
    )(x, codebooks, *planes)
    codes = jnp.transpose(codes_bqt, (1, 0, 2))
    commit_loss = jnp.sum(sq) / np.float32(_NQ * B * T * D)
    bw = jnp.asarray(
        _NQ * (np.log2(_K) * jnp.asarray(sample_rate).astype(jnp.float32) / 1000.0),
        dtype=jnp.float32)
    return q, codes, bw, commit_loss


# speedup vs baseline: 2.8230x; 1.3447x over previous
import jax
import jax.numpy as jnp
import numpy as np
from jax.experimental import pallas as pl
from jax.experimental.pallas import tpu as pltpu

_NQ = 8
_K = 1024
_D = 128
_TT = 512


def _rvq_kernel(x_ref, cb_ref, planes_ref, q_ref, codes_ref, sq_ref, c2_ref):
    @pl.when(jnp.logical_and(pl.program_id(0) == 0, pl.program_id(1) == 0))
    def _():
        for i in range(_NQ):
            cb = cb_ref[i]
            c2_ref[i, :] = jnp.sum(cb * cb, axis=1)

    xt = x_ref[0]
    r = xt.T
    codes_rows = []
    sq_rows = []
    iota = jax.lax.broadcasted_iota(jnp.int32, (_TT, _K), 1)
    for i in range(_NQ):
        cb = cb_ref[i]
        c2 = c2_ref[i, :]
        r2 = jnp.sum(r * r, axis=1, keepdims=True)
        dots = jax.lax.dot_general(
            r, cb, (((1,), (1,)), ((), ())),
            preferred_element_type=jnp.float32,
            precision=jax.lax.Precision.DEFAULT)
        dist = (r2 - 2.0 * dots) + c2[None, :]
        dmin = jnp.min(dist, axis=1, keepdims=True)
        idx = jnp.min(jnp.where(dist <= dmin, iota, _K), axis=1)
        codes_rows.append(idx)
        onehot = (iota == idx[:, None]).astype(jnp.bfloat16)
        bytes4 = jax.lax.dot_general(
            onehot, planes_ref[i], (((1,), (0,)), ((), ())),
            preferred_element_type=jnp.float32)
        bits = (bytes4[:, 0 * _D:1 * _D].astype(jnp.int32)
                | (bytes4[:, 1 * _D:2 * _D].astype(jnp.int32) << 8)
                | (bytes4[:, 2 * _D:3 * _D].astype(jnp.int32) << 16)
                | (bytes4[:, 3 * _D:4 * _D].astype(jnp.int32) << 24))
        q = jax.lax.bitcast_convert_type(bits, jnp.float32)
        r = r - q
        sq_rows.append(jnp.sum(r * r, axis=0))
    q_ref[0] = xt - r.T
    codes_ref[0] = jnp.stack(codes_rows, axis=0).astype(jnp.int32)
    sq_ref[0, 0] = jnp.stack(sq_rows, axis=0)


def kernel(x, sample_rate, codebooks):
    B, D, T = x.shape
    grid = (B, T // _TT)
    bits = jax.lax.bitcast_convert_type(codebooks, jnp.uint32)
    planes = jnp.concatenate(
        [((bits >> (8 * k)) & np.uint32(0xFF)).astype(jnp.bfloat16)
         for k in range(4)], axis=-1)
    q, codes_bqt, sq = pl.pallas_call(
        _rvq_kernel,
        grid=grid,
        in_specs=[
            pl.BlockSpec((1, D, _TT), lambda b, t: (b, 0, t)),
            pl.BlockSpec((_NQ, _K, _D), lambda b, t: (0, 0, 0)),
            pl.BlockSpec((_NQ, _K, 4 * _D), lambda b, t: (0, 0, 0)),
        ],
        out_specs=[
            pl.BlockSpec((1, D, _TT), lambda b, t: (b, 0, t)),
            pl.BlockSpec((1, _NQ, _TT), lambda b, t: (b, 0, t)),
            pl.BlockSpec((1, 1, _NQ, _D), lambda b, t: (b, t, 0, 0)),
        ],
        out_shape=[
            jax.ShapeDtypeStruct((B, D, T), jnp.float32),
            jax.ShapeDtypeStruct((B, _NQ, T), jnp.int32),
            jax.ShapeDtypeStruct((B, T // _TT, _NQ, _D), jnp.float32),
        ],
        scratch_shapes=[pltpu.VMEM((_NQ, _K), jnp.float32)],
    )(x, codebooks, planes)
    codes = jnp.transpose(codes_bqt, (1, 0, 2))
    commit_loss = jnp.sum(sq) / np.float32(_NQ * B * T * D)
    bw = jnp.asarray(
        _NQ * (np.log2(_K) * jnp.asarray(sample_rate).astype(jnp.float32) / 1000.0),
        dtype=jnp.float32)
    return q, codes, bw, commit_loss


# speedup vs baseline: 2.8255x; 1.0009x over previous
import jax
import jax.numpy as jnp
import numpy as np
from jax.experimental import pallas as pl

_NQ = 8
_K = 1024
_D = 128
_TC = 512
_TT = 1024


def _c2_kernel(cb_ref, c2_ref):
    for i in range(_NQ):
        cb = cb_ref[i]
        c2_ref[i, :] = jnp.sum(cb * cb, axis=1)


def _rvq_kernel(x_ref, cb_ref, c2_ref, planes_ref, q_ref, codes_ref, sq_ref):
    xt = x_ref[0]
    rs = [xt[:, :_TC].T, xt[:, _TC:].T]
    codes_rows = [[], []]
    sq_rows = [[], []]
    iota = jax.lax.broadcasted_iota(jnp.int32, (_TC, _K), 1)
    for i in range(_NQ):
        cb = cb_ref[i]
        c2 = c2_ref[i, :]
        for s in range(2):
            r = rs[s]
            r2 = jnp.sum(r * r, axis=1, keepdims=True)
            dots = jax.lax.dot_general(
                r, cb, (((1,), (1,)), ((), ())),
                preferred_element_type=jnp.float32,
                precision=jax.lax.Precision.DEFAULT)
            dist = (r2 - 2.0 * dots) + c2[None, :]
            dmin = jnp.min(dist, axis=1, keepdims=True)
            idx = jnp.min(jnp.where(dist <= dmin, iota, _K), axis=1)
            codes_rows[s].append(idx)
            onehot = (iota == idx[:, None]).astype(jnp.bfloat16)
            bytes4 = jax.lax.dot_general(
                onehot, planes_ref[i], (((1,), (0,)), ((), ())),
                preferred_element_type=jnp.float32)
            bits = (bytes4[:, 0 * _D:1 * _D].astype(jnp.int32)
                    | (bytes4[:, 1 * _D:2 * _D].astype(jnp.int32) << 8)
                    | (bytes4[:, 2 * _D:3 * _D].astype(jnp.int32) << 16)
                    | (bytes4[:, 3 * _D:4 * _D].astype(jnp.int32) << 24))
            q = jax.lax.bitcast_convert_type(bits, jnp.float32)
            rs[s] = r - q
            sq_rows[s].append(jnp.sum(rs[s] * rs[s], axis=0))
    q_ref[0] = xt - jnp.concatenate([rs[0].T, rs[1].T], axis=1)
    codes_ref[0] = jnp.concatenate(
        [jnp.stack(codes_rows[0], axis=0), jnp.stack(codes_rows[1], axis=0)],
        axis=1).astype(jnp.int32)
    sq_ref[0, 0] = (jnp.stack(sq_rows[0], axis=0)
                    + jnp.stack(sq_rows[1], axis=0))


def kernel(x, sample_rate, codebooks):
    B, D, T = x.shape
    grid = (B, T // _TT)
    bits = jax.lax.bitcast_convert_type(codebooks, jnp.uint32)
    planes = jnp.concatenate(
        [((bits >> (8 * k)) & np.uint32(0xFF)).astype(jnp.bfloat16)
         for k in range(4)], axis=-1)
    c2 = pl.pallas_call(
        _c2_kernel,
        out_shape=jax.ShapeDtypeStruct((_NQ, _K), jnp.float32),
    )(codebooks)
    q, codes_bqt, sq = pl.pallas_call(
        _rvq_kernel,
        grid=grid,
        in_specs=[
            pl.BlockSpec((1, D, _TT), lambda b, t: (b, 0, t)),
            pl.BlockSpec((_NQ, _K, _D), lambda b, t: (0, 0, 0)),
            pl.BlockSpec((_NQ, _K), lambda b, t: (0, 0)),
            pl.BlockSpec((_NQ, _K, 4 * _D), lambda b, t: (0, 0, 0)),
        ],
        out_specs=[
            pl.BlockSpec((1, D, _TT), lambda b, t: (b, 0, t)),
            pl.BlockSpec((1, _NQ, _TT), lambda b, t: (b, 0, t)),
            pl.BlockSpec((1, 1, _NQ, _D), lambda b, t: (b, t, 0, 0)),
        ],
        out_shape=[
            jax.ShapeDtypeStruct((B, D, T), jnp.float32),
            jax.ShapeDtypeStruct((B, _NQ, T), jnp.int32),
            jax.ShapeDtypeStruct((B, T // _TT, _NQ, _D), jnp.float32),
        ],
    )(x, codebooks, c2, planes)
    codes = jnp.transpose(codes_bqt, (1, 0, 2))
    commit_loss = jnp.sum(sq) / np.float32(_NQ * B * T * D)
    bw = jnp.asarray(
        _NQ * (np.log2(_K) * jnp.asarray(sample_rate).astype(jnp.float32) / 1000.0),
        dtype=jnp.float32)
    return q, codes, bw, commit_loss
